# baseline (speedup 1.0000x reference)
import functools

import jax
import jax.numpy as jnp
from jax.experimental import pallas as pl
from jax.experimental.pallas import tpu as pltpu

EMBED_DIM = 512
N_OUT = 256


def _mlp_block(x1_ref, x2_ref, w1_ref, b1_ref, w2_ref, b2_ref, out_ref):
    bf16 = jnp.bfloat16
    T = x1_ref.shape[1]
    xb = jnp.concatenate(
        (x1_ref[...].astype(bf16),
         x2_ref[...].astype(bf16),
         jnp.ones((1, T), bf16)),
        axis=0,
    )
    w1a = jnp.concatenate(
        (w1_ref[...].astype(bf16), b1_ref[...].astype(bf16)), axis=0
    )
    h = jax.lax.dot_general(
        xb, w1a, (((0,), (0,)), ((), ())),
        preferred_element_type=jnp.float32,
    ).astype(bf16)
    h = h * jax.nn.sigmoid(h)
    out_ref[...] = (
        jnp.dot(h, w2_ref[...].astype(bf16), preferred_element_type=jnp.float32)
        + b2_ref[...]
    )


@functools.partial(jax.jit, static_argnames=())
def kernel(pos1, pos2, W1, b1, W2, b2):
    B, S = pos1.shape
    N = B * S
    T = 2048
    grid = (N // T,)

    x1 = pos1.reshape(1, N)
    x2 = pos2.reshape(1, N)
    b1r = b1.reshape(1, EMBED_DIM)
    b2r = b2.reshape(1, N_OUT)

    row_spec = pl.BlockSpec((1, T), lambda i: (0, i))
    full = lambda shape: pl.BlockSpec(shape, lambda i: (0, 0))

    out = pl.pallas_call(
        _mlp_block,
        grid=grid,
        in_specs=[
            row_spec,
            row_spec,
            full((2, EMBED_DIM)),
            full((1, EMBED_DIM)),
            full((EMBED_DIM, N_OUT)),
            full((1, N_OUT)),
        ],
        out_specs=pl.BlockSpec((T, N_OUT), lambda i: (i, 0)),
        out_shape=jax.ShapeDtypeStruct((N, N_OUT), jnp.float32),
        compiler_params=pltpu.CompilerParams(
            dimension_semantics=("parallel",),
        ),
    )(x1, x2, W1, b1r, W2, b2r)
    return out.reshape(B, S, N_OUT)
